# baseline (speedup 1.0000x reference)
import functools

import jax
import jax.numpy as jnp
from jax import lax
from jax.experimental import pallas as pl
from jax.experimental.pallas import tpu as pltpu
from jax.experimental.pallas import tpu_sc as plsc

N = 10000
E = 160000
D = 256
H = 128
NC = 2
NS = 16
EPT = E // NS
K = 80
NB = EPT // K
KS = 104
NBS = 98
EPTP = KS * NBS
NP = N + 8
ZR = 40
NCH = N // ZR
CW = 16


def _for_my_chunks(s, fn):
    per = -(-NCH // NS)
    for j in range(per):
        if j < NCH - per * (NS - 1):
            fn(s * per + j)
        else:
            @pl.when(s < NS - 1)
            def _():
                fn(s * per + j)


def _zero_buf(buf, rows, width):
    def zb(i, _):
        buf[i // (width // 16), pl.ds((i % (width // 16)) * 16, 16)] = (
            jnp.zeros((16,), jnp.float32))
        return 0
    lax.fori_loop(0, rows * (width // 16), zb, 0)


def _seg_body(x2, src_hbm, dst_hbm, order_dep, out_s, src_v, dst_v,
              rows0, rows1, acc_sh, sem0, sem1):
    c = lax.axis_index("c")
    s = lax.axis_index("s")
    x2c = x2.at[c]
    rows = (rows0, rows1)
    sems = (sem0, sem1)
    zstage = rows0.at[pl.ds(0, ZR)]

    _zero_buf(zstage, ZR, H)
    _for_my_chunks(
        s, lambda ch: pltpu.sync_copy(zstage, acc_sh.at[pl.ds(ch * ZR, ZR)]))

    pltpu.sync_copy(src_hbm.at[s], src_v)
    pltpu.sync_copy(dst_hbm.at[s], dst_v)

    plsc.subcore_barrier()

    def sidx(b):
        return src_v.at[pl.ds(b * KS, KS)]

    pltpu.async_copy(x2c.at[sidx(0)], rows0, sem0)
    pltpu.async_copy(x2c.at[sidx(1)], rows1, sem1)

    @pl.loop(0, NBS, step=2)
    def _(b):
        for j in range(2):
            bj = b + j
            pltpu.make_async_copy(
                x2c.at[sidx(bj)], rows[j], sems[j]).wait()
            pltpu.sync_copy(rows[j], acc_sh.at[dst_v.at[bj]], add=True)

            @pl.when(bj + 2 < NBS)
            def _():
                pltpu.async_copy(x2c.at[sidx(bj + 2)], rows[j], sems[j])

    plsc.subcore_barrier()

    _for_my_chunks(
        s, lambda ch: pltpu.sync_copy(acc_sh.at[pl.ds(ch * ZR, ZR)],
                                      out_s.at[c].at[pl.ds(ch * ZR, ZR)]))


NB0 = NB // 2


def _cnt_body(ei_b, out_cnt, dst_v, ones_v, czero, cnt_sh, sem):
    c = lax.axis_index("c")
    s = lax.axis_index("s")

    _zero_buf(czero, ZR, H)
    _for_my_chunks(
        s, lambda ch: pltpu.sync_copy(czero, cnt_sh.at[pl.ds(ch * ZR, ZR)]))

    def ob(i, _):
        ones_v[i // 8, pl.ds((i % 8) * 16, 16)] = jnp.ones((16,), jnp.float32)
        return 0
    lax.fori_loop(0, K * 8, ob, 0)

    pltpu.sync_copy(ei_b.at[1, s], dst_v)

    plsc.subcore_barrier()

    lo = c * NB0
    hi = lo + NB0 + c * (NB - 2 * NB0)

    @pl.loop(lo, hi)
    def _(b):
        pltpu.sync_copy(ones_v, cnt_sh.at[dst_v.at[b]], add=True)

    plsc.subcore_barrier()

    _for_my_chunks(
        s, lambda ch: pltpu.sync_copy(cnt_sh.at[pl.ds(ch * ZR, ZR)],
                                      out_cnt.at[c].at[pl.ds(ch * ZR, ZR)]))


_sc_mesh = plsc.VectorSubcoreMesh(core_axis_name="c", subcore_axis_name="s")

_seg = pl.kernel(
    _seg_body,
    out_type=jax.ShapeDtypeStruct((NC, N, H), jnp.float32),
    mesh=_sc_mesh,
    scratch_types=[
        pltpu.VMEM((EPTP,), jnp.int32),
        pltpu.VMEM((NBS, KS), jnp.int32),
        pltpu.VMEM((KS, H), jnp.float32),
        pltpu.VMEM((KS, H), jnp.float32),
        pltpu.VMEM_SHARED((NP, H), jnp.float32),
        pltpu.SemaphoreType.DMA,
        pltpu.SemaphoreType.DMA,
    ],
)

_cnt = pl.kernel(
    _cnt_body,
    out_type=jax.ShapeDtypeStruct((NC, N, H), jnp.float32),
    mesh=_sc_mesh,
    scratch_types=[
        pltpu.VMEM((NB, K), jnp.int32),
        pltpu.VMEM((K, H), jnp.float32),
        pltpu.VMEM((ZR, H), jnp.float32),
        pltpu.VMEM_SHARED((N, H), jnp.float32),
        pltpu.SemaphoreType.DMA,
    ],
)


_R = 1000


def _hspec(i):
    return (0, i, 0)


def _selfmm_body(x_ref, w_ref, b_ref, out_ref):
    acc = (
        jnp.dot(x_ref[0], w_ref[0:H], preferred_element_type=jnp.float32)
        + jnp.dot(x_ref[1], w_ref[H:D], preferred_element_type=jnp.float32)
        + b_ref[...]
    )
    out_ref[0] = acc[:, 0:H]
    out_ref[1] = acc[:, H:D]


_selfmm = pl.pallas_call(
    _selfmm_body,
    grid=(N // _R,),
    in_specs=[
        pl.BlockSpec((NC, _R, H), _hspec),
        pl.BlockSpec((D, D), lambda i: (0, 0)),
        pl.BlockSpec((1, D), lambda i: (0, 0)),
    ],
    out_specs=pl.BlockSpec((NC, _R, H), _hspec),
    out_shape=jax.ShapeDtypeStruct((NC, N, H), jnp.float32),
)


def _pre_body(x_ref, w_ref, b_ref, x2_ref, xr_ref):
    x = x_ref[...]
    x2_ref[0] = x[:, 0:H]
    x2_ref[1] = x[:, H:D]
    acc = jnp.dot(x, w_ref[...], preferred_element_type=jnp.float32) + b_ref[...]
    xr_ref[0] = acc[:, 0:H]
    xr_ref[1] = acc[:, H:D]


_pre = pl.pallas_call(
    _pre_body,
    grid=(N // _R,),
    in_specs=[
        pl.BlockSpec((_R, D), lambda i: (i, 0)),
        pl.BlockSpec((D, D), lambda i: (0, 0)),
        pl.BlockSpec((1, D), lambda i: (0, 0)),
    ],
    out_specs=(pl.BlockSpec((NC, _R, H), _hspec),
               pl.BlockSpec((NC, _R, H), _hspec)),
    out_shape=(jax.ShapeDtypeStruct((NC, N, H), jnp.float32),
               jax.ShapeDtypeStruct((NC, N, H), jnp.float32)),
)


def _aggmm_body(last, s_ref, cnt_ref, r_ref, wl_ref, out_ref):
    cnt = jnp.maximum(cnt_ref[0, :, 0:1] + cnt_ref[1, :, 0:1], 1.0)
    acc = (
        jnp.dot(s_ref[0] / cnt, wl_ref[0:H], preferred_element_type=jnp.float32)
        + jnp.dot(s_ref[1] / cnt, wl_ref[H:D], preferred_element_type=jnp.float32)
        + jnp.concatenate([r_ref[0], r_ref[1]], axis=1)
    )
    if last:
        out_ref[...] = jax.nn.sigmoid(acc)
    else:
        h = jnp.maximum(acc, 0.0)
        out_ref[0] = h[:, 0:H]
        out_ref[1] = h[:, H:D]


def _make_aggmm(last):
    if last:
        out_shape = jax.ShapeDtypeStruct((N, D), jnp.float32)
        out_spec = pl.BlockSpec((_R, D), lambda i: (i, 0))
    else:
        out_shape = jax.ShapeDtypeStruct((NC, N, H), jnp.float32)
        out_spec = pl.BlockSpec((NC, _R, H), _hspec)
    return pl.pallas_call(
        functools.partial(_aggmm_body, last),
        grid=(N // _R,),
        in_specs=[
            pl.BlockSpec((NC, _R, H), _hspec),
            pl.BlockSpec((NC, _R, H), _hspec),
            pl.BlockSpec((NC, _R, H), _hspec),
            pl.BlockSpec((D, D), lambda i: (0, 0)),
        ],
        out_specs=out_spec,
        out_shape=out_shape,
    )


_aggmm_mid = _make_aggmm(False)
_aggmm_last = _make_aggmm(True)


def kernel(features, edge_index, W1l, b1, W1r, W2l, b2, W2r):
    ei_b = edge_index.reshape(2, NS, NB, K)
    src_p = jnp.concatenate(
        [edge_index[0].reshape(NS, EPT),
         jnp.zeros((NS, EPTP - EPT), jnp.int32)], axis=1)
    dst_p = jnp.concatenate(
        [edge_index[1].reshape(NS, EPT),
         jnp.full((NS, EPTP - EPT), N, jnp.int32)], axis=1
    ).reshape(NS, NBS, KS)
    b1r = b1.reshape(1, D)
    b2r = b2.reshape(1, D)

    cnt = _cnt(ei_b)
    x2, xr1 = _pre(features, W1r, b1r)
    s1 = _seg(x2, src_p, dst_p, cnt)
    h2 = _aggmm_mid(s1, cnt, xr1, W1l)
    hr2 = _selfmm(h2, W2r, b2r)
    s2 = _seg(h2, src_p, dst_p, s1)
    return _aggmm_last(s2, cnt, hr2, W2l)


# speedup vs baseline: 1.3107x; 1.3107x over previous
import functools

import jax
import jax.numpy as jnp
from jax import lax
from jax.experimental import pallas as pl
from jax.experimental.pallas import tpu as pltpu
from jax.experimental.pallas import tpu_sc as plsc

N = 10000
E = 160000
D = 256
H = 128
NC = 2
NS = 16
EPT = E // NS
K = 80
NB = EPT // K
KS = 96
NBS = 105
EPTP = KS * NBS
NP = N + 8
ZR = 40
NCH = N // ZR
CW = 16


def _for_my_chunks(s, fn):
    per = -(-NCH // NS)
    for j in range(per):
        if j < NCH - per * (NS - 1):
            fn(s * per + j)
        else:
            @pl.when(s < NS - 1)
            def _():
                fn(s * per + j)


def _zero_buf(buf, rows, width):
    def zb(i, _):
        buf[i // (width // 16), pl.ds((i % (width // 16)) * 16, 16)] = (
            jnp.zeros((16,), jnp.float32))
        return 0
    lax.fori_loop(0, rows * (width // 16), zb, 0)


def _seg_body(x2, src_hbm, dst_hbm, order_dep, out_s, src_v, dst_v,
              rows0, rows1, acc_sh, sem0, sem1):
    c = lax.axis_index("c")
    s = lax.axis_index("s")
    x2c = x2.at[c]
    rows = (rows0, rows1)
    sems = (sem0, sem1)
    zstage = rows0.at[pl.ds(0, ZR)]

    _zero_buf(zstage, ZR, H)
    _for_my_chunks(
        s, lambda ch: pltpu.sync_copy(zstage, acc_sh.at[pl.ds(ch * ZR, ZR)]))

    pltpu.sync_copy(src_hbm.at[s], src_v)
    pltpu.sync_copy(dst_hbm.at[s], dst_v)

    plsc.subcore_barrier()

    def sidx(b):
        return src_v.at[pl.ds(b * KS, KS)]

    pltpu.async_copy(x2c.at[sidx(0)], rows0, sem0)
    pltpu.async_copy(x2c.at[sidx(1)], rows1, sem1)

    @pl.loop(0, NBS - (NBS % 2), step=2)
    def _(b):
        for j in range(2):
            bj = b + j
            pltpu.make_async_copy(
                x2c.at[sidx(bj)], rows[j], sems[j]).wait()
            pltpu.sync_copy(rows[j], acc_sh.at[dst_v.at[bj]], add=True)

            @pl.when(bj + 2 < NBS)
            def _():
                pltpu.async_copy(x2c.at[sidx(bj + 2)], rows[j], sems[j])

    if NBS % 2:
        pltpu.make_async_copy(x2c.at[sidx(NBS - 1)], rows0, sem0).wait()
        pltpu.sync_copy(rows0, acc_sh.at[dst_v.at[NBS - 1]], add=True)

    plsc.subcore_barrier()

    _for_my_chunks(
        s, lambda ch: pltpu.sync_copy(acc_sh.at[pl.ds(ch * ZR, ZR)],
                                      out_s.at[c].at[pl.ds(ch * ZR, ZR)]))


NB0 = NB // 2


def _cnt_body(ei_b, out_cnt, dst_v, ones_v, czero, cnt_sh, sem):
    c = lax.axis_index("c")
    s = lax.axis_index("s")

    _zero_buf(czero, ZR, H)
    _for_my_chunks(
        s, lambda ch: pltpu.sync_copy(czero, cnt_sh.at[pl.ds(ch * ZR, ZR)]))

    def ob(i, _):
        ones_v[i // 8, pl.ds((i % 8) * 16, 16)] = jnp.ones((16,), jnp.float32)
        return 0
    lax.fori_loop(0, K * 8, ob, 0)

    pltpu.sync_copy(ei_b.at[1, s], dst_v)

    plsc.subcore_barrier()

    lo = c * NB0
    hi = lo + NB0 + c * (NB - 2 * NB0)

    @pl.loop(lo, hi)
    def _(b):
        pltpu.sync_copy(ones_v, cnt_sh.at[dst_v.at[b]], add=True)

    plsc.subcore_barrier()

    _for_my_chunks(
        s, lambda ch: pltpu.sync_copy(cnt_sh.at[pl.ds(ch * ZR, ZR)],
                                      out_cnt.at[c].at[pl.ds(ch * ZR, ZR)]))


_sc_mesh = plsc.VectorSubcoreMesh(core_axis_name="c", subcore_axis_name="s")

_seg = pl.kernel(
    _seg_body,
    out_type=jax.ShapeDtypeStruct((NC, N, H), jnp.float32),
    mesh=_sc_mesh,
    scratch_types=[
        pltpu.VMEM((EPTP,), jnp.int32),
        pltpu.VMEM((NBS, KS), jnp.int32),
        pltpu.VMEM((KS, H), jnp.float32),
        pltpu.VMEM((KS, H), jnp.float32),
        pltpu.VMEM_SHARED((NP, H), jnp.float32),
        pltpu.SemaphoreType.DMA,
        pltpu.SemaphoreType.DMA,
    ],
)

_cnt = pl.kernel(
    _cnt_body,
    out_type=jax.ShapeDtypeStruct((NC, N, H), jnp.float32),
    mesh=_sc_mesh,
    scratch_types=[
        pltpu.VMEM((NB, K), jnp.int32),
        pltpu.VMEM((K, H), jnp.float32),
        pltpu.VMEM((ZR, H), jnp.float32),
        pltpu.VMEM_SHARED((N, H), jnp.float32),
        pltpu.SemaphoreType.DMA,
    ],
)


_R = 1000


def _hspec(i):
    return (0, i, 0)


def _selfmm_body(x_ref, w_ref, b_ref, out_ref):
    acc = (
        jnp.dot(x_ref[0], w_ref[0:H], preferred_element_type=jnp.float32)
        + jnp.dot(x_ref[1], w_ref[H:D], preferred_element_type=jnp.float32)
        + b_ref[...]
    )
    out_ref[0] = acc[:, 0:H]
    out_ref[1] = acc[:, H:D]


_selfmm = pl.pallas_call(
    _selfmm_body,
    grid=(N // _R,),
    in_specs=[
        pl.BlockSpec((NC, _R, H), _hspec),
        pl.BlockSpec((D, D), lambda i: (0, 0)),
        pl.BlockSpec((1, D), lambda i: (0, 0)),
    ],
    out_specs=pl.BlockSpec((NC, _R, H), _hspec),
    out_shape=jax.ShapeDtypeStruct((NC, N, H), jnp.float32),
)


def _pre_body(x_ref, w_ref, b_ref, x2_ref, xr_ref):
    x = x_ref[...]
    x2_ref[0] = x[:, 0:H]
    x2_ref[1] = x[:, H:D]
    acc = jnp.dot(x, w_ref[...], preferred_element_type=jnp.float32) + b_ref[...]
    xr_ref[0] = acc[:, 0:H]
    xr_ref[1] = acc[:, H:D]


_pre = pl.pallas_call(
    _pre_body,
    grid=(N // _R,),
    in_specs=[
        pl.BlockSpec((_R, D), lambda i: (i, 0)),
        pl.BlockSpec((D, D), lambda i: (0, 0)),
        pl.BlockSpec((1, D), lambda i: (0, 0)),
    ],
    out_specs=(pl.BlockSpec((NC, _R, H), _hspec),
               pl.BlockSpec((NC, _R, H), _hspec)),
    out_shape=(jax.ShapeDtypeStruct((NC, N, H), jnp.float32),
               jax.ShapeDtypeStruct((NC, N, H), jnp.float32)),
)


def _aggmm_body(last, s_ref, cnt_ref, r_ref, wl_ref, out_ref):
    cnt = jnp.maximum(cnt_ref[0, :, 0:1] + cnt_ref[1, :, 0:1], 1.0)
    acc = (
        jnp.dot(s_ref[0] / cnt, wl_ref[0:H], preferred_element_type=jnp.float32)
        + jnp.dot(s_ref[1] / cnt, wl_ref[H:D], preferred_element_type=jnp.float32)
        + jnp.concatenate([r_ref[0], r_ref[1]], axis=1)
    )
    if last:
        out_ref[...] = jax.nn.sigmoid(acc)
    else:
        h = jnp.maximum(acc, 0.0)
        out_ref[0] = h[:, 0:H]
        out_ref[1] = h[:, H:D]


def _make_aggmm(last):
    if last:
        out_shape = jax.ShapeDtypeStruct((N, D), jnp.float32)
        out_spec = pl.BlockSpec((_R, D), lambda i: (i, 0))
    else:
        out_shape = jax.ShapeDtypeStruct((NC, N, H), jnp.float32)
        out_spec = pl.BlockSpec((NC, _R, H), _hspec)
    return pl.pallas_call(
        functools.partial(_aggmm_body, last),
        grid=(N // _R,),
        in_specs=[
            pl.BlockSpec((NC, _R, H), _hspec),
            pl.BlockSpec((NC, _R, H), _hspec),
            pl.BlockSpec((NC, _R, H), _hspec),
            pl.BlockSpec((D, D), lambda i: (0, 0)),
        ],
        out_specs=out_spec,
        out_shape=out_shape,
    )


_aggmm_mid = _make_aggmm(False)
_aggmm_last = _make_aggmm(True)


def kernel(features, edge_index, W1l, b1, W1r, W2l, b2, W2r):
    ei_b = edge_index.reshape(2, NS, NB, K)
    src_p = jnp.concatenate(
        [edge_index[0].reshape(NS, EPT),
         jnp.zeros((NS, EPTP - EPT), jnp.int32)], axis=1)
    dst_p = jnp.concatenate(
        [edge_index[1].reshape(NS, EPT),
         jnp.full((NS, EPTP - EPT), N, jnp.int32)], axis=1
    ).reshape(NS, NBS, KS)
    b1r = b1.reshape(1, D)
    b2r = b2.reshape(1, D)

    cnt = _cnt(ei_b)
    x2, xr1 = _pre(features, W1r, b1r)
    s1 = _seg(x2, src_p, dst_p, cnt)
    h2 = _aggmm_mid(s1, cnt, xr1, W1l)
    hr2 = _selfmm(h2, W2r, b2r)
    s2 = _seg(h2, src_p, dst_p, s1)
    return _aggmm_last(s2, cnt, hr2, W2l)


# speedup vs baseline: 1.6003x; 1.2209x over previous
import functools

import jax
import jax.numpy as jnp
from jax import lax
from jax.experimental import pallas as pl
from jax.experimental.pallas import tpu as pltpu
from jax.experimental.pallas import tpu_sc as plsc

N = 10000
E = 160000
D = 256
H = 128
NC = 2
NS = 16
EPT = E // NS
K = 80
NB = EPT // K
KS = 80
NBS = EPT // KS
EPTP = KS * NBS
NP = N + 8
ZR = 40
NCH = N // ZR
CW = 16


def _for_my_chunks(s, fn):
    per = -(-NCH // NS)
    for j in range(per):
        if j < NCH - per * (NS - 1):
            fn(s * per + j)
        else:
            @pl.when(s < NS - 1)
            def _():
                fn(s * per + j)


def _zero_buf(buf, rows, width):
    def zb(i, _):
        buf[i // (width // 16), pl.ds((i % (width // 16)) * 16, 16)] = (
            jnp.zeros((16,), jnp.float32))
        return 0
    lax.fori_loop(0, rows * (width // 16), zb, 0)


def _seg_body(x2, src_hbm, dst_hbm, order_dep, out_s, src_v, dst_v,
              rows0, rows1, acc_sh, sem0, sem1):
    c = lax.axis_index("c")
    s = lax.axis_index("s")
    x2c = x2.at[c]
    rows = (rows0, rows1)
    sems = (sem0, sem1)
    zstage = rows0.at[pl.ds(0, ZR)]

    _zero_buf(zstage, ZR, H)
    _for_my_chunks(
        s, lambda ch: pltpu.sync_copy(zstage, acc_sh.at[pl.ds(ch * ZR, ZR)]))

    pltpu.sync_copy(src_hbm.at[s], src_v)
    pltpu.sync_copy(dst_hbm.at[s], dst_v)

    plsc.subcore_barrier()

    def sidx(b):
        return src_v.at[pl.ds(b * KS, KS)]

    pltpu.async_copy(x2c.at[sidx(0)], rows0, sem0)
    pltpu.async_copy(x2c.at[sidx(1)], rows1, sem1)

    @pl.loop(0, NBS - (NBS % 2), step=2)
    def _(b):
        for j in range(2):
            bj = b + j
            pltpu.make_async_copy(
                x2c.at[sidx(bj)], rows[j], sems[j]).wait()
            pltpu.sync_copy(rows[j], acc_sh.at[dst_v.at[bj]], add=True)

            @pl.when(bj + 2 < NBS)
            def _():
                pltpu.async_copy(x2c.at[sidx(bj + 2)], rows[j], sems[j])

    if NBS % 2:
        pltpu.make_async_copy(x2c.at[sidx(NBS - 1)], rows0, sem0).wait()
        pltpu.sync_copy(rows0, acc_sh.at[dst_v.at[NBS - 1]], add=True)

    plsc.subcore_barrier()

    _for_my_chunks(
        s, lambda ch: pltpu.sync_copy(acc_sh.at[pl.ds(ch * ZR, ZR)],
                                      out_s.at[c].at[pl.ds(ch * ZR, ZR)]))


NB0 = NB // 2


def _cnt_body(ei_b, out_cnt, dst_v, ones_v, czero, cnt_sh, sem):
    c = lax.axis_index("c")
    s = lax.axis_index("s")

    _zero_buf(czero, ZR, H)
    _for_my_chunks(
        s, lambda ch: pltpu.sync_copy(czero, cnt_sh.at[pl.ds(ch * ZR, ZR)]))

    def ob(i, _):
        ones_v[i // 8, pl.ds((i % 8) * 16, 16)] = jnp.ones((16,), jnp.float32)
        return 0
    lax.fori_loop(0, K * 8, ob, 0)

    pltpu.sync_copy(ei_b.at[1, s], dst_v)

    plsc.subcore_barrier()

    lo = c * NB0
    hi = lo + NB0 + c * (NB - 2 * NB0)

    @pl.loop(lo, hi)
    def _(b):
        pltpu.sync_copy(ones_v, cnt_sh.at[dst_v.at[b]], add=True)

    plsc.subcore_barrier()

    _for_my_chunks(
        s, lambda ch: pltpu.sync_copy(cnt_sh.at[pl.ds(ch * ZR, ZR)],
                                      out_cnt.at[c].at[pl.ds(ch * ZR, ZR)]))


_sc_mesh = plsc.VectorSubcoreMesh(core_axis_name="c", subcore_axis_name="s")

_seg = pl.kernel(
    _seg_body,
    out_type=jax.ShapeDtypeStruct((NC, N, H), jnp.float32),
    mesh=_sc_mesh,
    scratch_types=[
        pltpu.VMEM((EPTP,), jnp.int32),
        pltpu.VMEM((NBS, KS), jnp.int32),
        pltpu.VMEM((KS, H), jnp.float32),
        pltpu.VMEM((KS, H), jnp.float32),
        pltpu.VMEM_SHARED((NP, H), jnp.float32),
        pltpu.SemaphoreType.DMA,
        pltpu.SemaphoreType.DMA,
    ],
)

_cnt = pl.kernel(
    _cnt_body,
    out_type=jax.ShapeDtypeStruct((NC, N, H), jnp.float32),
    mesh=_sc_mesh,
    scratch_types=[
        pltpu.VMEM((NB, K), jnp.int32),
        pltpu.VMEM((K, H), jnp.float32),
        pltpu.VMEM((ZR, H), jnp.float32),
        pltpu.VMEM_SHARED((N, H), jnp.float32),
        pltpu.SemaphoreType.DMA,
    ],
)


_R = 1000


def _hspec(i):
    return (0, i, 0)


def _selfmm_body(x_ref, w_ref, b_ref, out_ref):
    acc = (
        jnp.dot(x_ref[0], w_ref[0:H], preferred_element_type=jnp.float32)
        + jnp.dot(x_ref[1], w_ref[H:D], preferred_element_type=jnp.float32)
        + b_ref[...]
    )
    out_ref[0] = acc[:, 0:H]
    out_ref[1] = acc[:, H:D]


_selfmm = pl.pallas_call(
    _selfmm_body,
    grid=(N // _R,),
    in_specs=[
        pl.BlockSpec((NC, _R, H), _hspec),
        pl.BlockSpec((D, D), lambda i: (0, 0)),
        pl.BlockSpec((1, D), lambda i: (0, 0)),
    ],
    out_specs=pl.BlockSpec((NC, _R, H), _hspec),
    out_shape=jax.ShapeDtypeStruct((NC, N, H), jnp.float32),
)


def _pre_body(x_ref, w_ref, b_ref, x2_ref, xr_ref):
    x = x_ref[...]
    x2_ref[0] = x[:, 0:H]
    x2_ref[1] = x[:, H:D]
    acc = jnp.dot(x, w_ref[...], preferred_element_type=jnp.float32) + b_ref[...]
    xr_ref[0] = acc[:, 0:H]
    xr_ref[1] = acc[:, H:D]


_pre = pl.pallas_call(
    _pre_body,
    grid=(N // _R,),
    in_specs=[
        pl.BlockSpec((_R, D), lambda i: (i, 0)),
        pl.BlockSpec((D, D), lambda i: (0, 0)),
        pl.BlockSpec((1, D), lambda i: (0, 0)),
    ],
    out_specs=(pl.BlockSpec((NC, _R, H), _hspec),
               pl.BlockSpec((NC, _R, H), _hspec)),
    out_shape=(jax.ShapeDtypeStruct((NC, N, H), jnp.float32),
               jax.ShapeDtypeStruct((NC, N, H), jnp.float32)),
)


def _aggmm_body(last, s_ref, cnt_ref, r_ref, wl_ref, out_ref):
    cnt = jnp.maximum(cnt_ref[0, :, 0:1] + cnt_ref[1, :, 0:1], 1.0)
    acc = (
        jnp.dot(s_ref[0] / cnt, wl_ref[0:H], preferred_element_type=jnp.float32)
        + jnp.dot(s_ref[1] / cnt, wl_ref[H:D], preferred_element_type=jnp.float32)
        + jnp.concatenate([r_ref[0], r_ref[1]], axis=1)
    )
    if last:
        out_ref[...] = jax.nn.sigmoid(acc)
    else:
        h = jnp.maximum(acc, 0.0)
        out_ref[0] = h[:, 0:H]
        out_ref[1] = h[:, H:D]


def _make_aggmm(last):
    if last:
        out_shape = jax.ShapeDtypeStruct((N, D), jnp.float32)
        out_spec = pl.BlockSpec((_R, D), lambda i: (i, 0))
    else:
        out_shape = jax.ShapeDtypeStruct((NC, N, H), jnp.float32)
        out_spec = pl.BlockSpec((NC, _R, H), _hspec)
    return pl.pallas_call(
        functools.partial(_aggmm_body, last),
        grid=(N // _R,),
        in_specs=[
            pl.BlockSpec((NC, _R, H), _hspec),
            pl.BlockSpec((NC, _R, H), _hspec),
            pl.BlockSpec((NC, _R, H), _hspec),
            pl.BlockSpec((D, D), lambda i: (0, 0)),
        ],
        out_specs=out_spec,
        out_shape=out_shape,
    )


_aggmm_mid = _make_aggmm(False)
_aggmm_last = _make_aggmm(True)


def kernel(features, edge_index, W1l, b1, W1r, W2l, b2, W2r):
    ei_b = edge_index.reshape(2, NS, NB, K)
    src_p = edge_index[0].reshape(NS, EPTP)
    dst_p = edge_index[1].reshape(NS, NBS, KS)
    b1r = b1.reshape(1, D)
    b2r = b2.reshape(1, D)

    cnt = _cnt(ei_b)
    x2, xr1 = _pre(features, W1r, b1r)
    s1 = _seg(x2, src_p, dst_p, cnt)
    h2 = _aggmm_mid(s1, cnt, xr1, W1l)
    hr2 = _selfmm(h2, W2r, b2r)
    s2 = _seg(h2, src_p, dst_p, s1)
    return _aggmm_last(s2, cnt, hr2, W2l)
